# baseline (speedup 1.0000x reference)
import dataclasses

import jax
import jax.numpy as jnp
from jax import lax
from jax.experimental import pallas as pl
from jax.experimental.pallas import tpu as pltpu
from jax.experimental.pallas import tpu_sc as plsc

_B = 16384
_C = 1000
_NW = 32
_RPW = _B // _NW
_K = 16
_NBUF = 4
_NG = _RPW // _K


def _sc_body(idx_hbm, out_hbm, idx_v, buf0, buf1, buf2, buf3, sems):
    wid = lax.axis_index("s") * 2 + lax.axis_index("c")
    base = wid * _RPW

    pltpu.sync_copy(idx_hbm.at[pl.ds(base, _RPW)], idx_v)

    zeros16 = jnp.zeros((16,), jnp.float32)
    ones16 = jnp.full((16,), 1.0, jnp.float32)
    iota16 = lax.iota(jnp.int32, 16)

    bufs = (buf0, buf1, buf2, buf3)

    for buf in bufs:
        @pl.loop(0, _K)
        def _zero_row(k, buf=buf):
            for off in range(0, 992, 16):
                buf[k, pl.ds(off, 16)] = zeros16
            buf[k, pl.ds(984, 16)] = zeros16

    def scatter(buf, g, val16):
        idx16 = idx_v[pl.ds(g * _K, 16)]
        plsc.store_scatter(buf, [iota16, idx16], val16)

    for g in range(_NG):
        slot = g % _NBUF
        buf = bufs[slot]
        if g >= _NBUF:
            pltpu.make_async_copy(
                buf, out_hbm.at[pl.ds(base + (g - _NBUF) * _K, _K)], sems.at[slot]
            ).wait()
            scatter(buf, g - _NBUF, zeros16)
        scatter(buf, g, ones16)
        pltpu.async_copy(buf, out_hbm.at[pl.ds(base + g * _K, _K)], sems.at[slot])

    for g in range(_NG - _NBUF, _NG):
        slot = g % _NBUF
        pltpu.make_async_copy(
            bufs[slot], out_hbm.at[pl.ds(base + g * _K, _K)], sems.at[slot]
        ).wait()


def kernel(idxs):
    mesh = plsc.VectorSubcoreMesh(core_axis_name="c", subcore_axis_name="s")
    cp = pltpu.CompilerParams()
    if "needs_layout_passes" in pltpu.CompilerParams.__dataclass_fields__:
        cp = dataclasses.replace(cp, needs_layout_passes=False)
    sc_fn = pl.kernel(
        _sc_body,
        out_type=jax.ShapeDtypeStruct((_B, _C), jnp.float32),
        mesh=mesh,
        compiler_params=cp,
        scratch_types=[
            pltpu.VMEM((_RPW,), jnp.int32),
            pltpu.VMEM((_K, _C), jnp.float32),
            pltpu.VMEM((_K, _C), jnp.float32),
            pltpu.VMEM((_K, _C), jnp.float32),
            pltpu.VMEM((_K, _C), jnp.float32),
            pltpu.SemaphoreType.DMA((_NBUF,)),
        ],
    )
    return sc_fn(idxs.astype(jnp.int32))


# speedup vs baseline: 1.2423x; 1.2423x over previous
import jax
import jax.numpy as jnp
from jax.experimental import pallas as pl
from jax.experimental.pallas import tpu as pltpu

_B = 16384
_C = 1000
_BR = 4096
_NB = _B // _BR

def _onehot_block(idx_ref, out_ref):
    idx = idx_ref[0, 0, :].reshape(_BR, 1)
    cols = jax.lax.broadcasted_iota(jnp.int32, out_ref.shape, 1)
    out_ref[...] = jnp.where(idx == cols, 1.0, 0.0)

def kernel(idxs):
    idxs3 = idxs.astype(jnp.int32).reshape(_NB, 1, _BR)
    return pl.pallas_call(
        _onehot_block,
        grid=(_NB,),
        in_specs=[pl.BlockSpec((1, 1, _BR), lambda i: (i, 0, 0))],
        out_specs=pl.BlockSpec((_BR, _C), lambda i: (i, 0)),
        out_shape=jax.ShapeDtypeStruct((_B, _C), jnp.float32),
        compiler_params=pltpu.CompilerParams(
            dimension_semantics=("parallel",),
        ),
    )(idxs3)
